# baseline (speedup 1.0000x reference)
import functools

import jax
import jax.numpy as jnp
from jax import lax
from jax.experimental import pallas as pl
from jax.experimental.pallas import tpu as pltpu
from jax.experimental.pallas import tpu_sc as plsc

H = 32
NE1 = 1537
NSP = 512
NTAB = 15 * NE1 + NSP
NTAB_PAD = 23680
B, N = 16, 64
EL = B * N * N
NW = 32
EPW = EL // NW
CHUNK = 64
NIDX = CHUNK * 16
NPAIR = EPW // (2 * CHUNK)


def _prep_body(enc0_ref, enc1_ref, enc2_ref, w_ref, spw_ref, fidx_ref,
               out_ref, oidx_ref):
    encs = (enc0_ref[...], enc1_ref[...], enc2_ref[...])
    for k in range(15):
        d, j = k // 3, k % 3
        t = jnp.dot(encs[j], w_ref[d], preferred_element_type=jnp.float32)
        out_ref[pl.ds(k * NE1, NE1), :] = t * (1.0 / 3.0)
    s = lax.broadcasted_iota(jnp.int32, (NSP, 1), 0)
    sp = jnp.where(s == 0, 1, s)
    sp = jnp.where(sp > 1, sp - 1, sp)
    sp = jnp.minimum(sp, 5)
    out_ref[pl.ds(15 * NE1, NSP), :] = spw_ref[...] * sp.astype(jnp.float32)
    out_ref[pl.ds(NTAB, NTAB_PAD - NTAB), :] = jnp.zeros(
        (NTAB_PAD - NTAB, H), jnp.float32)
    off = (lax.broadcasted_iota(jnp.int32, (1, 128), 1) % 16) * NE1
    oidx_ref[...] = fidx_ref[...] + off


def _build_table(enc0, enc1, enc2, w5, spatial_w, fidx):
    return pl.pallas_call(
        _prep_body,
        out_shape=[
            jax.ShapeDtypeStruct((NTAB_PAD, H), jnp.float32),
            jax.ShapeDtypeStruct((EL // 8, 128), jnp.int32),
        ],
    )(enc0, enc1, enc2, w5, spatial_w, fidx)


def _sc_body(tab_hbm, fidx_hbm, out_hbm, idx_a, idx_b, rows_a, rows_b,
             zeros_v, dst_a, dst_b, acc_sh, tab_sh, sem_a, sem_b):
    sid = lax.axis_index("s")
    wid = sid * 2 + lax.axis_index("c")
    a0 = pl.multiple_of(sid * 2 * CHUNK, CHUNK)
    b0 = pl.multiple_of(sid * 2 * CHUNK + CHUNK, CHUNK)
    t0 = pl.multiple_of(sid * (NTAB_PAD // 16), 8)
    pltpu.sync_copy(tab_hbm.at[pl.ds(t0, NTAB_PAD // 16)],
                    tab_sh.at[pl.ds(t0, NTAB_PAD // 16)])
    plsc.subcore_barrier()
    z = jnp.zeros((16,), jnp.float32)
    zi = jnp.zeros((16,), jnp.int32)
    for e in range(CHUNK):
        dst_a[pl.ds(e * 16, 16)] = zi + (sid * 2 * CHUNK + e)
        dst_b[pl.ds(e * 16, 16)] = zi + (sid * 2 * CHUNK + CHUNK + e)
    for e in range(2 * CHUNK):
        zeros_v[e, pl.ds(0, 16)] = z
        zeros_v[e, pl.ds(16, 16)] = z

    def pair_body(p, carry):
        ea = pl.multiple_of(wid * EPW + (2 * p) * CHUNK, CHUNK)
        eb = pl.multiple_of(wid * EPW + (2 * p + 1) * CHUNK, CHUNK)
        pltpu.sync_copy(fidx_hbm.at[pl.ds(pl.multiple_of(ea * 16, NIDX), NIDX)],
                        idx_a)
        cpa = pltpu.async_copy(tab_sh.at[idx_a], rows_a, sem_a)
        pltpu.sync_copy(fidx_hbm.at[pl.ds(pl.multiple_of(eb * 16, NIDX), NIDX)],
                        idx_b)
        cpb = pltpu.async_copy(tab_sh.at[idx_b], rows_b, sem_b)
        pltpu.sync_copy(zeros_v, acc_sh.at[pl.ds(a0, 2 * CHUNK)])
        cpa.wait()
        pltpu.sync_copy(rows_a, acc_sh.at[dst_a], add=True)
        pltpu.sync_copy(acc_sh.at[pl.ds(a0, CHUNK)], out_hbm.at[pl.ds(ea, CHUNK)])
        cpb.wait()
        pltpu.sync_copy(rows_b, acc_sh.at[dst_b], add=True)
        pltpu.sync_copy(acc_sh.at[pl.ds(b0, CHUNK)], out_hbm.at[pl.ds(eb, CHUNK)])
        return carry

    lax.fori_loop(0, NPAIR, pair_body, 0)


def _sc_gather(tab, fidx):
    mesh = plsc.VectorSubcoreMesh(core_axis_name="c", subcore_axis_name="s",
                                  num_cores=2, num_subcores=16)
    fn = pl.kernel(
        _sc_body,
        out_type=jax.ShapeDtypeStruct((EL, H), jnp.float32),
        mesh=mesh,
        compiler_params=pltpu.CompilerParams(use_tc_tiling_on_sc=False,
                                             needs_layout_passes=False),
        scratch_types=[
            pltpu.VMEM((NIDX,), jnp.int32),
            pltpu.VMEM((NIDX,), jnp.int32),
            pltpu.VMEM((NIDX, H), jnp.float32),
            pltpu.VMEM((NIDX, H), jnp.float32),
            pltpu.VMEM((2 * CHUNK, H), jnp.float32),
            pltpu.VMEM((NIDX,), jnp.int32),
            pltpu.VMEM((NIDX,), jnp.int32),
            pltpu.VMEM_SHARED((16 * 2 * CHUNK, H), jnp.float32),
            pltpu.VMEM_SHARED((NTAB_PAD, H), jnp.float32),
            pltpu.SemaphoreType.DMA,
            pltpu.SemaphoreType.DMA,
        ],
    )
    return fn(tab, fidx)


def _asm_body(ab_ref, sp_ref, u_ref, virt_ref, out_ref):
    s = sp_ref[0, 0]
    sp = jnp.where(s == 0, 1, s)
    sp = jnp.where(sp > 1, sp - 1, sp)
    sp = jnp.minimum(sp, 5)
    u = u_ref[0]
    ut = u.T.reshape(H, N, N)
    u3 = ut * (1.0 / sp.astype(jnp.float32))[None]
    ab = ab_ref[0]
    vb = virt_ref[...].reshape(H, 1)
    out_ref[0, :, 0, :] = ab[0:1, :] + vb
    out_ref[0, :, 1:, 0:1] = ab[1:, 0:1][None, :, :] + vb.reshape(H, 1, 1)
    out_ref[0, :, 1:, 1:] = u3 + ab[1:, 1:][None]


def _assemble(attn_bias, spatial_sq, unscaled, virt_w):
    return pl.pallas_call(
        _asm_body,
        grid=(B,),
        in_specs=[
            pl.BlockSpec((1, N + 1, N + 1), lambda b: (b, 0, 0)),
            pl.BlockSpec((1, 1, N, N), lambda b: (b, 0, 0, 0)),
            pl.BlockSpec((1, N * N, H), lambda b: (b, 0, 0)),
            pl.BlockSpec((1, H), lambda b: (0, 0)),
        ],
        out_specs=pl.BlockSpec((1, H, N + 1, N + 1), lambda b: (b, 0, 0, 0)),
        out_shape=jax.ShapeDtypeStruct((B, H, N + 1, N + 1), jnp.float32),
    )(attn_bias, spatial_sq, unscaled, virt_w)


def kernel(attn_bias, spatial_pos, edge_input, edge_enc0, edge_enc1,
           edge_enc2, edge_dis_w, spatial_w, virt_w):
    spi = spatial_pos.astype(jnp.int32)
    ei = edge_input.astype(jnp.int32).reshape(EL, 15)
    fidx = jnp.concatenate([ei, spi.reshape(EL, 1)], axis=1)
    fidx = fidx.reshape(EL // 8, 128)
    w5 = edge_dis_w.reshape(-1)[: 5 * H * H].reshape(5, H, H)

    tab, oidx = _build_table(edge_enc0, edge_enc1, edge_enc2, w5, spatial_w,
                             fidx)
    unscaled = _sc_gather(tab, oidx.reshape(EL * 16))
    return _assemble(attn_bias, spi.reshape(B, 1, N, N),
                     unscaled.reshape(B, N * N, H), virt_w)
